# Initial kernel scaffold; baseline (speedup 1.0000x reference)
import jax
import jax.numpy as jnp
from jax.experimental import pallas as pl


def kernel(features, lengths):
    raise NotImplementedError("write your pallas kernel here")


# speedup vs baseline: 68.0250x; 68.0250x over previous
import functools

import jax
import jax.numpy as jnp
from jax import lax
from jax.experimental import pallas as pl
from jax.experimental.pallas import tpu as pltpu
from jax.experimental.pallas import tpu_sc as plsc

B, T, C = 16, 2048, 512
L = 16
NC, NS = 2, 16
NW = NC * NS
WPB = NW // B
CPW = C // WPB
NG = CPW // L
R = 128

_NEG = float("-inf")

_mesh = plsc.VectorSubcoreMesh(core_axis_name="c", subcore_axis_name="s")


@functools.partial(
    pl.kernel,
    mesh=_mesh,
    out_type=jax.ShapeDtypeStruct((B, C), jnp.float32),
    scratch_types=[
        pltpu.VMEM((L,), jnp.int32),
        pltpu.VMEM((R, CPW), jnp.float32),
        pltpu.VMEM((CPW,), jnp.float32),
    ],
)
def _masked_max(feat_hbm, len_hbm, out_hbm, len_v, buf_v, res_v):
    wid = lax.axis_index("s") * NC + lax.axis_index("c")
    b = wid // WPB
    c0 = (wid % WPB) * CPW

    pltpu.sync_copy(len_hbm.at[b], len_v)
    mylen = len_v[...][0]

    nblk = (mylen + R - 1) // R

    def block_body(i, acc):
        pltpu.sync_copy(feat_hbm.at[b, pl.ds(i * R, R), pl.ds(c0, CPW)], buf_v)
        nrows = jnp.minimum(mylen - i * R, R)

        def row_body(r, acc):
            return tuple(
                jnp.maximum(acc[g], buf_v[r, pl.ds(g * L, L)])
                for g in range(NG)
            )

        return lax.fori_loop(0, nrows, row_body, acc)

    acc0 = tuple(jnp.full((L,), _NEG, jnp.float32) for _ in range(NG))
    acc = lax.fori_loop(0, nblk, block_body, acc0)

    nonzero = mylen > 0
    full = mylen >= T
    for g in range(NG):
        v = jnp.where(full, acc[g], jnp.maximum(acc[g], jnp.float32(-10000.0)))
        res_v[pl.ds(g * L, L)] = jnp.where(nonzero, v, jnp.float32(0.0))

    pltpu.sync_copy(res_v, out_hbm.at[b, pl.ds(c0, CPW)])


def kernel(features, lengths):
    len_bcast = jnp.broadcast_to(lengths.astype(jnp.int32)[:, None], (B, L))
    return _masked_max(features, len_bcast)


# speedup vs baseline: 110.7673x; 1.6283x over previous
import functools

import jax
import jax.numpy as jnp
from jax import lax
from jax.experimental import pallas as pl
from jax.experimental.pallas import tpu as pltpu
from jax.experimental.pallas import tpu_sc as plsc

B, T, C = 16, 2048, 512
L = 16
NC, NS = 2, 16
NW = NC * NS
WPB = NW // B
CPW = C // WPB
NG = CPW // L
R = 128

_NEG = float("-inf")

_mesh = plsc.VectorSubcoreMesh(core_axis_name="c", subcore_axis_name="s")


@functools.partial(
    pl.kernel,
    mesh=_mesh,
    out_type=jax.ShapeDtypeStruct((B, C), jnp.float32),
    scratch_types=[
        pltpu.VMEM((L,), jnp.int32),
        pltpu.VMEM((R, CPW), jnp.float32),
        pltpu.VMEM((R, CPW), jnp.float32),
        pltpu.VMEM((CPW,), jnp.float32),
        pltpu.SemaphoreType.DMA,
        pltpu.SemaphoreType.DMA,
    ],
)
def _masked_max(feat_hbm, len_hbm, out_hbm, len_v, buf0, buf1, res_v,
                sem0, sem1):
    wid = lax.axis_index("s") * NC + lax.axis_index("c")
    b = wid // WPB
    c0 = (wid % WPB) * CPW

    pltpu.sync_copy(len_hbm.at[b], len_v)
    mylen = len_v[...][0]

    nblk = (mylen + R - 1) // R
    bufs = (buf0, buf1)
    sems = (sem0, sem1)

    def copy(i, k):
        return pltpu.make_async_copy(
            feat_hbm.at[b, pl.ds(i * R, R), pl.ds(c0, CPW)], bufs[k], sems[k])

    @pl.when(nblk > 0)
    def _():
        copy(0, 0).start()

    @pl.when(nblk > 1)
    def _():
        copy(1, 1).start()

    def step(i, k, acc):
        @pl.when(i < nblk)
        def _():
            copy(i, k).wait()

        @pl.when(i + 2 < nblk)
        def _():
            copy(i + 2, k).start()

        nrows = jnp.maximum(jnp.minimum(mylen - i * R, R), 0)
        buf = bufs[k]

        def row_body(r, acc):
            return tuple(
                jnp.maximum(acc[g], buf[r, pl.ds(g * L, L)])
                for g in range(NG)
            )

        return lax.fori_loop(0, nrows, row_body, acc)

    def pair_body(j, acc):
        acc = step(2 * j, 0, acc)
        return step(2 * j + 1, 1, acc)

    acc0 = tuple(jnp.full((L,), _NEG, jnp.float32) for _ in range(NG))
    acc = lax.fori_loop(0, (nblk + 1) // 2, pair_body, acc0)

    nonzero = mylen > 0
    full = mylen >= T
    for g in range(NG):
        v = jnp.where(full, acc[g], jnp.maximum(acc[g], jnp.float32(-10000.0)))
        res_v[pl.ds(g * L, L)] = jnp.where(nonzero, v, jnp.float32(0.0))

    pltpu.sync_copy(res_v, out_hbm.at[b, pl.ds(c0, CPW)])


def kernel(features, lengths):
    len_bcast = jnp.broadcast_to(lengths.astype(jnp.int32)[:, None], (B, L))
    return _masked_max(features, len_bcast)
